# baseline (speedup 1.0000x reference)
import functools

import jax
import jax.numpy as jnp
import numpy as np
from jax import lax
from jax.experimental import pallas as pl
from jax.experimental.pallas import tpu as pltpu
from jax.experimental.pallas import tpu_sc as plsc

NB_CODE = 512
CODE_DIM = 512
WIDTH = 512
DEPTH = 3
DOWN_T = 3
DRATE = 3
INPUT_DIM = 251
N_TOK = 2048

PAD = 16
NROW = N_TOK + 2 * PAD
K3 = 3 * WIDTH

_NC = 2
_NS = 16
_NW = _NC * _NS
_B_PER_W = N_TOK // _NW

_DILS = tuple(DRATE ** d for d in range(DEPTH))[::-1]

_K_PLAIN_X = 0
_K_PLAIN_OUT = 1
_K_RES1 = 2
_K_RES2 = 3


def _sc_gather(codebook, idx):
    mesh = plsc.VectorSubcoreMesh(core_axis_name="c", subcore_axis_name="s")

    @functools.partial(
        pl.kernel,
        out_type=jax.ShapeDtypeStruct((N_TOK, CODE_DIM), jnp.float32),
        mesh=mesh,
        scratch_types=[
            pltpu.VMEM((_B_PER_W,), jnp.int32),
            pltpu.VMEM((_B_PER_W, CODE_DIM), jnp.float32),
            pltpu.SemaphoreType.DMA,
        ],
    )
    def gather_kernel(table_hbm, idx_hbm, out_hbm, idx_v, rows_v, sem):
        wid = lax.axis_index("s") * _NC + lax.axis_index("c")
        base = wid * _B_PER_W
        pltpu.sync_copy(idx_hbm.at[pl.ds(base, _B_PER_W)], idx_v)
        pltpu.async_copy(table_hbm.at[idx_v], rows_v, sem).wait()
        pltpu.sync_copy(rows_v, out_hbm.at[pl.ds(base, _B_PER_W)])

    return gather_kernel(codebook, idx)


def _layer_schedule():
    layers = [(_K_PLAIN_X, 1, 1)]
    for _ in range(DOWN_T):
        for dil in _DILS:
            layers.append((_K_RES1, dil, 0))
            layers.append((_K_RES2, 1, 0))
        layers.append((_K_PLAIN_X, 1, 0))
    layers.append((_K_PLAIN_X, 1, 1))
    layers.append((_K_PLAIN_OUT, 1, 0))
    return layers


_LAYERS = _layer_schedule()
_N_LAYERS = len(_LAYERS)
_KINDS = [k for (k, _, _) in _LAYERS]
_CTRL = np.asarray([[k, d, p] for (k, d, p) in _LAYERS], dtype=np.int32)


def _param_list(params):
    out = [(params['conv_in']['w'], params['conv_in']['b'])]
    for blk in params['blocks']:
        for rb in blk['res']:
            out.append((rb['c1']['w'], rb['c1']['b']))
            out.append((rb['c2']['w'], rb['c2']['b']))
        out.append((blk['conv']['w'], blk['conv']['b']))
    out.append((params['conv_mid']['w'], params['conv_mid']['b']))
    out.append((params['conv_out']['w'], params['conv_out']['b']))
    return out


def _raw_weights(params):
    ws, biases = [], []
    for (w, b) in _param_list(params):
        o, c, k = w.shape
        w2 = w.reshape(o, c * k)
        if o % 8 != 0:
            w2 = jnp.pad(w2, ((0, -o % 8), (0, 0)))
        ws.append(w2)
        if o < WIDTH:
            b = jnp.pad(b, (0, WIDTH - o))
        biases.append(b)
    return ws, jnp.stack(biases)


_SLAB_ROWS = [WIDTH] * _N_LAYERS
_SLAB_ROWS[-1] = INPUT_DIM + (-INPUT_DIM % 8)
_SLAB_COLS = [WIDTH if k == _K_RES2 else K3 for k in _KINDS]


def _layer_body(*args):
    ctrl_ref, g_ref = args[0], args[1]
    w_refs = args[2:2 + _N_LAYERS]
    b_ref = args[2 + _N_LAYERS]
    out_ref = args[3 + _N_LAYERS]
    x_ref, t_ref, cat_ref, slab_ref, wbuf_ref, sem = args[4 + _N_LAYERS:]

    i = pl.program_id(0)
    kind = ctrl_ref[i, 0]
    dil = ctrl_ref[i, 1]
    post = ctrl_ref[i, 2]

    def dma(l, slot):
        rows, cols = _SLAB_ROWS[l], _SLAB_COLS[l]
        return pltpu.make_async_copy(
            w_refs[l], slab_ref.at[slot, pl.ds(0, rows), pl.ds(0, cols)], sem)

    def prep(l_kind, slot):
        def _prep_k3():
            a = slab_ref[slot]
            at3 = jnp.swapaxes(a, 0, 1).reshape(WIDTH, 3, WIDTH)
            for j in range(3):
                wbuf_ref[slot, pl.ds(j * WIDTH, WIDTH), :] = at3[:, j, :]

        def _prep_k1():
            a = slab_ref[slot, :, :WIDTH]
            wbuf_ref[slot, pl.ds(0, WIDTH), :] = jnp.swapaxes(a, 0, 1)

        if isinstance(l_kind, int):
            _prep_k1() if l_kind == _K_RES2 else _prep_k3()
        else:
            pl.when(l_kind != _K_RES2)(_prep_k3)
            pl.when(l_kind == _K_RES2)(_prep_k1)

    @pl.when(i == 0)
    def _init():
        x_ref[...] = jnp.zeros((NROW, WIDTH), jnp.float32)
        t_ref[...] = jnp.zeros((NROW, WIDTH), jnp.float32)
        x_ref[PAD:PAD + N_TOK, :] = g_ref[...]
        dma(0, 0).start()
        dma(0, 0).wait()
        prep(_KINDS[0], 0)
        dma(1, 1).start()

    @pl.when(i < _N_LAYERS - 1)
    def _pipeline():
        nxt = i + 1
        for l in range(1, _N_LAYERS):
            @pl.when(nxt == l)
            def _(l=l):
                dma(l, l % 2).wait()
                if l + 1 < _N_LAYERS:
                    dma(l + 1, (l + 1) % 2).start()
        prep(ctrl_ref[nxt, 0], nxt % 2)

    slot = i % 2
    bias = b_ref[pl.ds(i, 1), :]

    def finish_plain(val):
        val = jnp.where(post == 1, jnp.maximum(val, 0.0), val)

        @pl.when(kind == _K_PLAIN_X)
        def _():
            x_ref[PAD:PAD + N_TOK, :] = val

        @pl.when(kind == _K_PLAIN_OUT)
        def _():
            out_ref[...] = val

    def build_cat(src_ref, d, with_relu):
        for j, s in enumerate((-d, 0, d)):
            v = src_ref[PAD + s:PAD + s + N_TOK, :]
            if with_relu:
                v = jnp.maximum(v, 0.0)
            cat_ref[:, j * WIDTH:(j + 1) * WIDTH] = v

    @pl.when(kind <= _K_PLAIN_OUT)
    def _plain():
        build_cat(x_ref, 1, False)
        val = jnp.dot(cat_ref[...], wbuf_ref[slot],
                      preferred_element_type=jnp.float32) + bias
        finish_plain(val)

    for d in _DILS:
        @pl.when((kind == _K_RES1) & (dil == d))
        def _res1(d=d):
            build_cat(x_ref, d, True)
            t_ref[PAD:PAD + N_TOK, :] = jnp.dot(
                cat_ref[...], wbuf_ref[slot], preferred_element_type=jnp.float32
            ) + bias

    @pl.when(kind == _K_RES2)
    def _res2():
        v = jnp.maximum(t_ref[PAD:PAD + N_TOK, :], 0.0)
        val = jnp.dot(v, wbuf_ref[slot, :WIDTH, :],
                      preferred_element_type=jnp.float32) + bias
        x_ref[PAD:PAD + N_TOK, :] += val


def _tc_decode(g, raw_ws, big_b):
    ctrl = jnp.asarray(_CTRL)
    return pl.pallas_call(
        _layer_body,
        grid=(_N_LAYERS,),
        in_specs=[
            pl.BlockSpec(memory_space=pltpu.SMEM),
            pl.BlockSpec((N_TOK, WIDTH), lambda i: (0, 0)),
        ] + [pl.BlockSpec(memory_space=pl.ANY)] * _N_LAYERS + [
            pl.BlockSpec((_N_LAYERS, WIDTH), lambda i: (0, 0)),
        ],
        out_specs=pl.BlockSpec((N_TOK, WIDTH), lambda i: (0, 0)),
        out_shape=jax.ShapeDtypeStruct((N_TOK, WIDTH), jnp.float32),
        scratch_shapes=[
            pltpu.VMEM((NROW, WIDTH), jnp.float32),
            pltpu.VMEM((NROW, WIDTH), jnp.float32),
            pltpu.VMEM((N_TOK, K3), jnp.float32),
            pltpu.VMEM((2, WIDTH, K3), jnp.float32),
            pltpu.VMEM((2, K3, WIDTH), jnp.float32),
            pltpu.SemaphoreType.DMA,
        ],
        compiler_params=pltpu.CompilerParams(
            dimension_semantics=("arbitrary",),
        ),
    )(ctrl, g, *raw_ws, big_b)


def kernel(x, codebook, params):
    idx = x.astype(jnp.int32)
    g = _sc_gather(codebook, idx)
    raw_ws, big_b = _raw_weights(params)
    out = _tc_decode(g, raw_ws, big_b)
    return out[:, :INPUT_DIM].reshape(1, N_TOK, INPUT_DIM)


# speedup vs baseline: 1.3261x; 1.3261x over previous
import functools

import jax
import jax.numpy as jnp
import numpy as np
from jax import lax
from jax.experimental import pallas as pl
from jax.experimental.pallas import tpu as pltpu
from jax.experimental.pallas import tpu_sc as plsc

NB_CODE = 512
CODE_DIM = 512
WIDTH = 512
DEPTH = 3
DOWN_T = 3
DRATE = 3
INPUT_DIM = 251
N_TOK = 2048

PAD = 16
NROW = N_TOK + 2 * PAD
K3 = 3 * WIDTH

_NC = 2
_NS = 16
_NW = _NC * _NS
_B_PER_W = N_TOK // _NW

_DILS = tuple(DRATE ** d for d in range(DEPTH))[::-1]

_K_PLAIN_X = 0
_K_PLAIN_OUT = 1
_K_RES1 = 2
_K_RES2 = 3


def _sc_gather(codebook, idx):
    mesh = plsc.VectorSubcoreMesh(core_axis_name="c", subcore_axis_name="s")

    @functools.partial(
        pl.kernel,
        out_type=jax.ShapeDtypeStruct((N_TOK, CODE_DIM), jnp.float32),
        mesh=mesh,
        scratch_types=[
            pltpu.VMEM((_B_PER_W,), jnp.int32),
            pltpu.VMEM((_B_PER_W, CODE_DIM), jnp.float32),
            pltpu.SemaphoreType.DMA,
        ],
    )
    def gather_kernel(table_hbm, idx_hbm, out_hbm, idx_v, rows_v, sem):
        wid = lax.axis_index("s") * _NC + lax.axis_index("c")
        base = wid * _B_PER_W
        pltpu.sync_copy(idx_hbm.at[pl.ds(base, _B_PER_W)], idx_v)
        pltpu.async_copy(table_hbm.at[idx_v], rows_v, sem).wait()
        pltpu.sync_copy(rows_v, out_hbm.at[pl.ds(base, _B_PER_W)])

    return gather_kernel(codebook, idx)


_RES_TRIPLE = []
for _d in _DILS:
    _RES_TRIPLE += [(_K_RES1, _d, 0), (_K_RES2, 1, 0)]
_STAGES = [
    [(_K_PLAIN_X, 1, 1)] + _RES_TRIPLE + [(_K_PLAIN_OUT, 1, 0)],
    _RES_TRIPLE + [(_K_PLAIN_OUT, 1, 0)],
    _RES_TRIPLE + [(_K_PLAIN_OUT, 1, 0)],
    [(_K_PLAIN_X, 1, 1), (_K_PLAIN_OUT, 1, 0)],
]
_STAGE_SLICES = [(0, 8), (8, 15), (15, 22), (22, 24)]


def _param_list(params):
    out = [(params['conv_in']['w'], params['conv_in']['b'])]
    for blk in params['blocks']:
        for rb in blk['res']:
            out.append((rb['c1']['w'], rb['c1']['b']))
            out.append((rb['c2']['w'], rb['c2']['b']))
        out.append((blk['conv']['w'], blk['conv']['b']))
    out.append((params['conv_mid']['w'], params['conv_mid']['b']))
    out.append((params['conv_out']['w'], params['conv_out']['b']))
    return out


def _pack_stage(plist, sched):
    k3_ws, k1_ws, biases = [], [], []
    for (w, b), (kind, _, _) in zip(plist, sched):
        o = w.shape[0]
        if o < WIDTH:
            w = jnp.pad(w, ((0, WIDTH - o), (0, 0), (0, 0)))
            b = jnp.pad(b, (0, WIDTH - o))
        if kind == _K_RES2:
            k1_ws.append(w[:, :, 0])
        else:
            k3_ws.append(w)
        biases.append(b)
    big3 = jnp.stack(k3_ws).transpose(0, 3, 2, 1).reshape(len(k3_ws), K3, WIDTH)
    if k1_ws:
        big1 = jnp.stack(k1_ws).transpose(0, 2, 1)
    else:
        big1 = jnp.zeros((1, WIDTH, WIDTH), jnp.float32)
    return big3, big1, jnp.stack(biases)


def _make_body(sched):
    n_layers = len(sched)

    def body(ctrl_ref, xin_ref, w3_ref, w1_ref, b_ref, out_ref,
             x_ref, t_ref, cat_ref):
        i = pl.program_id(0)
        kind = ctrl_ref[i, 0]
        dil = ctrl_ref[i, 1]
        post = ctrl_ref[i, 2]

        @pl.when(i == 0)
        def _init():
            x_ref[...] = jnp.zeros((NROW, WIDTH), jnp.float32)
            t_ref[...] = jnp.zeros((NROW, WIDTH), jnp.float32)
            x_ref[PAD:PAD + N_TOK, :] = xin_ref[...]

        bias = b_ref[pl.ds(i, 1), :]

        def finish_plain(val):
            val = jnp.where(post == 1, jnp.maximum(val, 0.0), val)

            @pl.when(kind == _K_PLAIN_X)
            def _():
                x_ref[PAD:PAD + N_TOK, :] = val

            @pl.when(kind == _K_PLAIN_OUT)
            def _():
                out_ref[...] = val

        def build_cat(src_ref, d, with_relu):
            for j, s in enumerate((-d, 0, d)):
                v = src_ref[PAD + s:PAD + s + N_TOK, :]
                if with_relu:
                    v = jnp.maximum(v, 0.0)
                cat_ref[:, j * WIDTH:(j + 1) * WIDTH] = v

        @pl.when(kind <= _K_PLAIN_OUT)
        def _plain():
            build_cat(x_ref, 1, False)
            val = jnp.dot(cat_ref[...], w3_ref[0],
                          preferred_element_type=jnp.float32) + bias
            finish_plain(val)

        for d in set(d for (k, d, _) in sched if k == _K_RES1):
            @pl.when((kind == _K_RES1) & (dil == d))
            def _res1(d=d):
                build_cat(x_ref, d, True)
                t_ref[PAD:PAD + N_TOK, :] = jnp.dot(
                    cat_ref[...], w3_ref[0],
                    preferred_element_type=jnp.float32) + bias

        if any(k == _K_RES2 for (k, _, _) in sched):
            @pl.when(kind == _K_RES2)
            def _res2():
                v = jnp.maximum(t_ref[PAD:PAD + N_TOK, :], 0.0)
                val = jnp.dot(v, w1_ref[0],
                              preferred_element_type=jnp.float32) + bias
                x_ref[PAD:PAD + N_TOK, :] += val

    return body


def _stage_call(sched, xin, big3, big1, biases):
    n_layers = len(sched)
    n3 = big3.shape[0]
    n1 = big1.shape[0]
    lead = 1 if sched[0][0] == _K_PLAIN_X and n_layers > 2 else 0
    if n_layers == 2:
        k3_map = lambda i: (jnp.minimum(i, n3 - 1), 0, 0)
        k1_map = lambda i: (0, 0, 0)
    else:
        k3_map = lambda i: (jnp.minimum((i + 1) // 2, n3 - 1), 0, 0)
        k1_map = lambda i: (
            jnp.clip((i - lead) // 2, 0, n1 - 1), 0, 0)

    ctrl = jnp.asarray(
        np.asarray([[k, d, p] for (k, d, p) in sched], dtype=np.int32))
    return pl.pallas_call(
        _make_body(sched),
        grid=(n_layers,),
        in_specs=[
            pl.BlockSpec(memory_space=pltpu.SMEM),
            pl.BlockSpec((N_TOK, WIDTH), lambda i: (0, 0)),
            pl.BlockSpec((1, K3, WIDTH), k3_map),
            pl.BlockSpec((1, WIDTH, WIDTH), k1_map),
            pl.BlockSpec((n_layers, WIDTH), lambda i: (0, 0)),
        ],
        out_specs=pl.BlockSpec((N_TOK, WIDTH), lambda i: (0, 0)),
        out_shape=jax.ShapeDtypeStruct((N_TOK, WIDTH), jnp.float32),
        scratch_shapes=[
            pltpu.VMEM((NROW, WIDTH), jnp.float32),
            pltpu.VMEM((NROW, WIDTH), jnp.float32),
            pltpu.VMEM((N_TOK, K3), jnp.float32),
        ],
        compiler_params=pltpu.CompilerParams(
            dimension_semantics=("arbitrary",),
        ),
    )(ctrl, xin, big3, big1, biases)


def kernel(x, codebook, params):
    idx = x.astype(jnp.int32)
    h = _sc_gather(codebook, idx)
    plist = _param_list(params)
    for sched, (lo, hi) in zip(_STAGES, _STAGE_SLICES):
        big3, big1, biases = _pack_stage(plist[lo:hi], sched)
        h = _stage_call(sched, h, big3, big1, biases)
    return h[:, :INPUT_DIM].reshape(1, N_TOK, INPUT_DIM)


# speedup vs baseline: 1.4943x; 1.1269x over previous
import functools

import jax
import jax.numpy as jnp
import numpy as np
from jax import lax
from jax.experimental import pallas as pl
from jax.experimental.pallas import tpu as pltpu
from jax.experimental.pallas import tpu_sc as plsc

NB_CODE = 512
CODE_DIM = 512
WIDTH = 512
DEPTH = 3
DOWN_T = 3
DRATE = 3
INPUT_DIM = 251
N_TOK = 2048

PAD = 16
NROW = N_TOK + 2 * PAD
K3 = 3 * WIDTH

_NC = 2
_NS = 16
_NW = _NC * _NS
_B_PER_W = N_TOK // _NW

_DILS = tuple(DRATE ** d for d in range(DEPTH))[::-1]

_K_PLAIN_X = 0
_K_PLAIN_OUT = 1
_K_RES1 = 2
_K_RES2 = 3


def _sc_gather(codebook, idx):
    mesh = plsc.VectorSubcoreMesh(core_axis_name="c", subcore_axis_name="s")

    @functools.partial(
        pl.kernel,
        out_type=jax.ShapeDtypeStruct((N_TOK, CODE_DIM), jnp.float32),
        mesh=mesh,
        scratch_types=[
            pltpu.VMEM((_B_PER_W,), jnp.int32),
            pltpu.VMEM((_B_PER_W, CODE_DIM), jnp.float32),
            pltpu.SemaphoreType.DMA,
        ],
    )
    def gather_kernel(table_hbm, idx_hbm, out_hbm, idx_v, rows_v, sem):
        wid = lax.axis_index("s") * _NC + lax.axis_index("c")
        base = wid * _B_PER_W
        pltpu.sync_copy(idx_hbm.at[pl.ds(base, _B_PER_W)], idx_v)
        pltpu.async_copy(table_hbm.at[idx_v], rows_v, sem).wait()
        pltpu.sync_copy(rows_v, out_hbm.at[pl.ds(base, _B_PER_W)])

    return gather_kernel(codebook, idx)


def _layer_schedule():
    layers = [(_K_PLAIN_X, 1, 1)]
    for _ in range(DOWN_T):
        for dil in _DILS:
            layers.append((_K_RES1, dil, 0))
            layers.append((_K_RES2, 1, 0))
        layers.append((_K_PLAIN_X, 1, 0))
    layers.append((_K_PLAIN_X, 1, 1))
    layers.append((_K_PLAIN_OUT, 1, 0))
    return layers


_LAYERS = _layer_schedule()
_N_LAYERS = len(_LAYERS)
_CTRL = np.asarray([[k, d, p] for (k, d, p) in _LAYERS], dtype=np.int32)

_N_K3 = sum(1 for (k, _, _) in _LAYERS if k != _K_RES2)
_N_K1 = _N_LAYERS - _N_K3


def _k3_block_index(i):
    b = (i - 1) // 7
    r = (i - 1) % 7
    return jnp.minimum(1 + 4 * b + (r + 1) // 2, _N_K3 - 1)


def _k1_block_index(i):
    b = (i - 1) // 7
    r = (i - 1) % 7
    return jnp.minimum(3 * b + r // 2, _N_K1 - 1)


def _param_list(params):
    out = [(params['conv_in']['w'], params['conv_in']['b'])]
    for blk in params['blocks']:
        for rb in blk['res']:
            out.append((rb['c1']['w'], rb['c1']['b']))
            out.append((rb['c2']['w'], rb['c2']['b']))
        out.append((blk['conv']['w'], blk['conv']['b']))
    out.append((params['conv_mid']['w'], params['conv_mid']['b']))
    out.append((params['conv_out']['w'], params['conv_out']['b']))
    return out


def _pack_weights(params):
    k3_ws, k1_ws, biases = [], [], []
    for (w, b), (kind, _, _) in zip(_param_list(params), _LAYERS):
        o = w.shape[0]
        if o < WIDTH:
            w = jnp.pad(w, ((0, WIDTH - o), (0, 0), (0, 0)))
            b = jnp.pad(b, (0, WIDTH - o))
        if kind == _K_RES2:
            k1_ws.append(w[:, :, 0])
        else:
            k3_ws.append(w)
        biases.append(b)
    big3 = jnp.stack(k3_ws).transpose(0, 3, 2, 1).reshape(_N_K3, K3, WIDTH)
    big1 = jnp.stack(k1_ws).transpose(0, 2, 1)
    return big3, big1, jnp.stack(biases)


def _layer_body(ctrl_ref, g_ref, w3_ref, w1_ref, b_ref, out_ref,
                x_ref, t_ref, cat_ref):
    i = pl.program_id(0)
    kind = ctrl_ref[i, 0]
    dil = ctrl_ref[i, 1]
    post = ctrl_ref[i, 2]

    @pl.when(i == 0)
    def _init():
        x_ref[...] = jnp.zeros((NROW, WIDTH), jnp.float32)
        t_ref[...] = jnp.zeros((NROW, WIDTH), jnp.float32)
        x_ref[PAD:PAD + N_TOK, :] = g_ref[...]

    bias = b_ref[pl.ds(i, 1), :]

    CH = N_TOK // 4

    def build_chunk(r0, d, with_relu):
        for j, s in enumerate((-d, 0, d)):
            v = x_ref[PAD + s + r0:PAD + s + r0 + CH, :]
            if with_relu:
                v = jnp.maximum(v, 0.0)
            cat_ref[r0:r0 + CH, j * WIDTH:(j + 1) * WIDTH] = v

    def dot_chunk(r0):
        return jnp.dot(cat_ref[r0:r0 + CH, :], w3_ref[0],
                       preferred_element_type=jnp.float32) + bias

    def conv3_chunked(d, with_relu, emit, writes_x):
        if writes_x:
            for c in range(4):
                build_chunk(c * CH, d, with_relu)
            for c in range(4):
                emit(dot_chunk(c * CH), c * CH)
        else:
            for c in range(4):
                build_chunk(c * CH, d, with_relu)
                emit(dot_chunk(c * CH), c * CH)

    def emit_plain(val, r0):
        val = jnp.where(post == 1, jnp.maximum(val, 0.0), val)

        @pl.when(kind == _K_PLAIN_X)
        def _():
            x_ref[PAD + r0:PAD + r0 + CH, :] = val

        @pl.when(kind == _K_PLAIN_OUT)
        def _():
            out_ref[r0:r0 + CH, :] = val

    @pl.when(kind <= _K_PLAIN_OUT)
    def _plain():
        conv3_chunked(1, False, emit_plain, writes_x=True)

    def emit_t(val, r0):
        t_ref[PAD + r0:PAD + r0 + CH, :] = val

    for d in _DILS:
        @pl.when((kind == _K_RES1) & (dil == d))
        def _res1(d=d):
            conv3_chunked(d, True, emit_t, writes_x=False)

    @pl.when(kind == _K_RES2)
    def _res2():
        for c in range(4):
            r0 = c * CH
            v = jnp.maximum(t_ref[PAD + r0:PAD + r0 + CH, :], 0.0)
            val = jnp.dot(v, w1_ref[0],
                          preferred_element_type=jnp.float32) + bias
            x_ref[PAD + r0:PAD + r0 + CH, :] += val


def _tc_decode(g, big3, big1, big_b):
    ctrl = jnp.asarray(_CTRL)
    return pl.pallas_call(
        _layer_body,
        grid=(_N_LAYERS,),
        in_specs=[
            pl.BlockSpec(memory_space=pltpu.SMEM),
            pl.BlockSpec((N_TOK, WIDTH), lambda i: (0, 0)),
            pl.BlockSpec((1, K3, WIDTH), lambda i: (_k3_block_index(i), 0, 0)),
            pl.BlockSpec((1, WIDTH, WIDTH), lambda i: (_k1_block_index(i), 0, 0)),
            pl.BlockSpec((_N_LAYERS, WIDTH), lambda i: (0, 0)),
        ],
        out_specs=pl.BlockSpec((N_TOK, WIDTH), lambda i: (0, 0)),
        out_shape=jax.ShapeDtypeStruct((N_TOK, WIDTH), jnp.float32),
        scratch_shapes=[
            pltpu.VMEM((NROW, WIDTH), jnp.float32),
            pltpu.VMEM((NROW, WIDTH), jnp.float32),
            pltpu.VMEM((N_TOK, K3), jnp.float32),
        ],
        compiler_params=pltpu.CompilerParams(
            dimension_semantics=("arbitrary",),
        ),
    )(ctrl, g, big3, big1, big_b)


def kernel(x, codebook, params):
    idx = x.astype(jnp.int32)
    g = _sc_gather(codebook, idx)
    big3, big1, big_b = _pack_weights(params)
    out = _tc_decode(g, big3, big1, big_b)
    return out[:, :INPUT_DIM].reshape(1, N_TOK, INPUT_DIM)


# speedup vs baseline: 1.5653x; 1.0475x over previous
import functools

import jax
import jax.numpy as jnp
import numpy as np
from jax import lax
from jax.experimental import pallas as pl
from jax.experimental.pallas import tpu as pltpu
from jax.experimental.pallas import tpu_sc as plsc

NB_CODE = 512
CODE_DIM = 512
WIDTH = 512
DEPTH = 3
DOWN_T = 3
DRATE = 3
INPUT_DIM = 251
N_TOK = 2048

PAD = 16
NROW = N_TOK + 2 * PAD
K3 = 3 * WIDTH

_NC = 2
_NS = 16
_NW = _NC * _NS
_B_PER_W = N_TOK // _NW

_DILS = tuple(DRATE ** d for d in range(DEPTH))[::-1]

_K_PLAIN_X = 0
_K_PLAIN_OUT = 1
_K_RES1 = 2
_K_RES2 = 3


def _sc_gather(codebook, idx):
    mesh = plsc.VectorSubcoreMesh(core_axis_name="c", subcore_axis_name="s")

    @functools.partial(
        pl.kernel,
        out_type=jax.ShapeDtypeStruct((N_TOK, CODE_DIM), jnp.float32),
        mesh=mesh,
        scratch_types=[
            pltpu.VMEM((_B_PER_W,), jnp.int32),
            pltpu.VMEM((_B_PER_W, CODE_DIM), jnp.float32),
            pltpu.SemaphoreType.DMA,
        ],
    )
    def gather_kernel(table_hbm, idx_hbm, out_hbm, idx_v, rows_v, sem):
        wid = lax.axis_index("s") * _NC + lax.axis_index("c")
        base = wid * _B_PER_W
        pltpu.sync_copy(idx_hbm.at[pl.ds(base, _B_PER_W)], idx_v)
        pltpu.async_copy(table_hbm.at[idx_v], rows_v, sem).wait()
        pltpu.sync_copy(rows_v, out_hbm.at[pl.ds(base, _B_PER_W)])

    return gather_kernel(codebook, idx)


def _layer_schedule():
    layers = [(_K_PLAIN_X, 1, 1)]
    for _ in range(DOWN_T):
        for dil in _DILS:
            layers.append((_K_RES1, dil, 0))
            layers.append((_K_RES2, 1, 0))
        layers.append((_K_PLAIN_X, 1, 0))
    layers.append((_K_PLAIN_X, 1, 1))
    layers.append((_K_PLAIN_OUT, 1, 0))
    return layers


_LAYERS = _layer_schedule()
_N_LAYERS = len(_LAYERS)
_CTRL = np.asarray([[k, d, p] for (k, d, p) in _LAYERS], dtype=np.int32)

_N_K3 = sum(1 for (k, _, _) in _LAYERS if k != _K_RES2)
_N_K1 = _N_LAYERS - _N_K3


def _k3_block_index(i):
    b = (i - 1) // 7
    r = (i - 1) % 7
    return jnp.minimum(1 + 4 * b + (r + 1) // 2, _N_K3 - 1)


def _k1_block_index(i):
    b = (i - 1) // 7
    r = (i - 1) % 7
    return jnp.minimum(3 * b + r // 2, _N_K1 - 1)


def _param_list(params):
    out = [(params['conv_in']['w'], params['conv_in']['b'])]
    for blk in params['blocks']:
        for rb in blk['res']:
            out.append((rb['c1']['w'], rb['c1']['b']))
            out.append((rb['c2']['w'], rb['c2']['b']))
        out.append((blk['conv']['w'], blk['conv']['b']))
    out.append((params['conv_mid']['w'], params['conv_mid']['b']))
    out.append((params['conv_out']['w'], params['conv_out']['b']))
    return out


def _pack_weights(params):
    k3_ws, k1_ws, biases = [], [], []
    for (w, b), (kind, _, _) in zip(_param_list(params), _LAYERS):
        o = w.shape[0]
        if o < WIDTH:
            w = jnp.pad(w, ((0, WIDTH - o), (0, 0), (0, 0)))
            b = jnp.pad(b, (0, WIDTH - o))
        if kind == _K_RES2:
            k1_ws.append(w[:, :, 0])
        else:
            k3_ws.append(w)
        biases.append(b)
    big3 = jnp.stack(k3_ws).transpose(0, 3, 2, 1).reshape(_N_K3, K3, WIDTH)
    big1 = jnp.stack(k1_ws).transpose(0, 2, 1)
    return big3, big1, jnp.stack(biases)


def _layer_body(ctrl_ref, g_ref, w3_ref, w1_ref, b_ref, out_ref,
                x_ref, t_ref, cat_ref):
    i = pl.program_id(0)
    kind = ctrl_ref[i, 0]
    dil = ctrl_ref[i, 1]
    post = ctrl_ref[i, 2]

    @pl.when(i == 0)
    def _init():
        x_ref[...] = jnp.zeros((NROW, WIDTH), jnp.float32)
        t_ref[...] = jnp.zeros((NROW, WIDTH), jnp.float32)
        x_ref[PAD:PAD + N_TOK, :] = g_ref[...]

    bias = b_ref[pl.ds(i, 1), :]

    def finish_plain(val):
        val = jnp.where(post == 1, jnp.maximum(val, 0.0), val)

        @pl.when(kind == _K_PLAIN_X)
        def _():
            x_ref[PAD:PAD + N_TOK, :] = val

        @pl.when(kind == _K_PLAIN_OUT)
        def _():
            out_ref[...] = val

    def build_cat(src_ref, d, with_relu):
        for j, s in enumerate((-d, 0, d)):
            v = src_ref[PAD + s:PAD + s + N_TOK, :]
            if with_relu:
                v = jnp.maximum(v, 0.0)
            cat_ref[:, j * WIDTH:(j + 1) * WIDTH] = v

    @pl.when(kind <= _K_PLAIN_OUT)
    def _plain():
        build_cat(x_ref, 1, False)
        val = jnp.dot(cat_ref[...], w3_ref[0],
                      preferred_element_type=jnp.float32) + bias
        finish_plain(val)

    for d in _DILS:
        @pl.when((kind == _K_RES1) & (dil == d))
        def _res1(d=d):
            build_cat(x_ref, d, True)
            t_ref[PAD:PAD + N_TOK, :] = jnp.dot(
                cat_ref[...], w3_ref[0], preferred_element_type=jnp.float32
            ) + bias

    @pl.when(kind == _K_RES2)
    def _res2():
        v = jnp.maximum(t_ref[PAD:PAD + N_TOK, :], 0.0)
        val = jnp.dot(v, w1_ref[0],
                      preferred_element_type=jnp.float32) + bias
        x_ref[PAD:PAD + N_TOK, :] += val


def _tc_decode(g, big3, big1, big_b):
    ctrl = jnp.asarray(_CTRL)
    return pl.pallas_call(
        _layer_body,
        grid=(_N_LAYERS,),
        in_specs=[
            pl.BlockSpec(memory_space=pltpu.SMEM),
            pl.BlockSpec((N_TOK, WIDTH), lambda i: (0, 0)),
            pl.BlockSpec((1, K3, WIDTH), lambda i: (_k3_block_index(i), 0, 0)),
            pl.BlockSpec((1, WIDTH, WIDTH), lambda i: (_k1_block_index(i), 0, 0)),
            pl.BlockSpec((_N_LAYERS, WIDTH), lambda i: (0, 0)),
        ],
        out_specs=pl.BlockSpec((N_TOK, WIDTH), lambda i: (0, 0)),
        out_shape=jax.ShapeDtypeStruct((N_TOK, WIDTH), jnp.float32),
        scratch_shapes=[
            pltpu.VMEM((NROW, WIDTH), jnp.float32),
            pltpu.VMEM((NROW, WIDTH), jnp.float32),
            pltpu.VMEM((N_TOK, K3), jnp.float32),
        ],
        compiler_params=pltpu.CompilerParams(
            dimension_semantics=("arbitrary",),
        ),
    )(ctrl, g, big3, big1, big_b)


def kernel(x, codebook, params):
    idx = x.astype(jnp.int32)
    g = _sc_gather(codebook, idx)
    big3, big1, big_b = _pack_weights(params)
    out = _tc_decode(g, big3, big1, big_b)
    return out[:, :INPUT_DIM].reshape(1, N_TOK, INPUT_DIM)
